# Initial kernel scaffold; baseline (speedup 1.0000x reference)
import jax
import jax.numpy as jnp
from jax.experimental import pallas as pl


def kernel(hidden, target, noise_samples, weight, bias, logprob_noise):
    raise NotImplementedError("write your pallas kernel here")


# speedup vs baseline: 1.9635x; 1.9635x over previous
import functools

import jax
import jax.numpy as jnp
from jax import lax
from jax.experimental import pallas as pl
from jax.experimental.pallas import tpu as pltpu, tpu_sc as plsc


def _sc_gather(aug, noise_ids, target_ids):
    V, HA = aug.shape
    K = noise_ids.shape[0]
    T = target_ids.shape[0]
    info = plsc.get_sparse_core_info()
    NC, NS = info.num_cores, info.num_subcores
    NW = NC * NS
    kn, kt = K // NW, T // NW
    RC = 64

    mesh = plsc.VectorSubcoreMesh(core_axis_name="c", subcore_axis_name="s")

    @functools.partial(
        pl.kernel,
        mesh=mesh,
        out_type=(
            jax.ShapeDtypeStruct((K, HA), jnp.float32),
            jax.ShapeDtypeStruct((T, HA), jnp.float32),
        ),
        scratch_types=[
            pltpu.VMEM((RC,), jnp.int32),
            pltpu.VMEM((RC, HA), jnp.float32),
            pltpu.SemaphoreType.DMA,
        ],
    )
    def sc_kernel(aug_hbm, nid_hbm, tid_hbm, nw_hbm, ew_hbm,
                  idx_v, rows_v, sem):
        wid = lax.axis_index("s") * NC + lax.axis_index("c")
        nbase = wid * kn
        tbase = wid * kt

        for c in range(kn // RC):
            pltpu.sync_copy(nid_hbm.at[pl.ds(nbase + c * RC, RC)], idx_v)
            pltpu.async_copy(aug_hbm.at[idx_v], rows_v, sem).wait()
            pltpu.sync_copy(rows_v, nw_hbm.at[pl.ds(nbase + c * RC, RC)])

        for c in range(kt // RC):
            pltpu.sync_copy(tid_hbm.at[pl.ds(tbase + c * RC, RC)], idx_v)
            pltpu.async_copy(aug_hbm.at[idx_v], rows_v, sem).wait()
            pltpu.sync_copy(rows_v, ew_hbm.at[pl.ds(tbase + c * RC, RC)])

    return sc_kernel(aug, noise_ids, target_ids)


def _cast_bf16(nw_aug, H):
    K, HA = nw_aug.shape
    NB = 8

    def body(in_ref, out_ref):
        out_ref[...] = in_ref[:, :H].astype(jnp.bfloat16)

    return pl.pallas_call(
        body,
        grid=(NB,),
        in_specs=[pl.BlockSpec((K // NB, HA), lambda i: (i, 0))],
        out_specs=pl.BlockSpec((K // NB, H), lambda i: (i, 0)),
        out_shape=jax.ShapeDtypeStruct((K, H), jnp.bfloat16),
    )(nw_aug)


def _tc_main(x, nw16, nb2, lpn2, ew_aug):
    T, H = x.shape
    K = nw16.shape[0]
    HA = ew_aug.shape[1]
    TT = 256

    def body(x_ref, nw_ref, nb_ref, lpn_ref, ew_ref,
             out4_ref, out1_ref, out3_ref, out2_ref):
        xb = x_ref[...]
        acc = lax.dot_general(
            xb.astype(jnp.bfloat16), nw_ref[...],
            (((1,), (1,)), ((), ())),
            preferred_element_type=jnp.float32)
        out4_ref[...] = acc + nb_ref[...]
        out1_ref[...] = jnp.broadcast_to(lpn_ref[...], (TT, K))
        ew = ew_ref[...]
        out3_ref[...] = jnp.sum(xb * ew[:, :H], axis=1, keepdims=True) \
            + ew[:, H:H + 1]
        out2_ref[...] = ew[:, H + 1:H + 2]

    return pl.pallas_call(
        body,
        grid=(T // TT,),
        in_specs=[
            pl.BlockSpec((TT, H), lambda i: (i, 0)),
            pl.BlockSpec((K, H), lambda i: (0, 0)),
            pl.BlockSpec((1, K), lambda i: (0, 0)),
            pl.BlockSpec((1, K), lambda i: (0, 0)),
            pl.BlockSpec((TT, HA), lambda i: (i, 0)),
        ],
        out_specs=[
            pl.BlockSpec((TT, K), lambda i: (i, 0)),
            pl.BlockSpec((TT, K), lambda i: (i, 0)),
            pl.BlockSpec((TT, 1), lambda i: (i, 0)),
            pl.BlockSpec((TT, 1), lambda i: (i, 0)),
        ],
        out_shape=[
            jax.ShapeDtypeStruct((T, K), jnp.float32),
            jax.ShapeDtypeStruct((T, K), jnp.float32),
            jax.ShapeDtypeStruct((T, 1), jnp.float32),
            jax.ShapeDtypeStruct((T, 1), jnp.float32),
        ],
    )(x, nw16, nb2, lpn2, ew_aug)


def kernel(hidden, target, noise_samples, weight, bias, logprob_noise):
    seq_len, bsz, H = hidden.shape
    T = seq_len * bsz
    K = noise_samples.shape[0]
    V = weight.shape[0]
    x = hidden.reshape(T, H)
    tgt = target.reshape(T)

    pad = 128 - 2
    aug = jnp.concatenate(
        [weight, bias[:, None], logprob_noise[:, None],
         jnp.zeros((V, pad), jnp.float32)], axis=1)

    nw_aug, ew_aug = _sc_gather(aug, noise_samples, tgt)
    nw16 = _cast_bf16(nw_aug, H)
    nb2 = nw_aug[:, H].reshape(1, K)
    lpn2 = nw_aug[:, H + 1].reshape(1, K)
    out4, out1, out3, out2 = _tc_main(x, nw16, nb2, lpn2, ew_aug)
    return (out1, out2, out3, out4)
